# Initial kernel scaffold; baseline (speedup 1.0000x reference)
import jax
import jax.numpy as jnp
from jax.experimental import pallas as pl


def kernel(x, gW1, gb1, gW2, gb2, We1, be1, We2, be2):
    raise NotImplementedError("write your pallas kernel here")


# speedup vs baseline: 3.2615x; 3.2615x over previous
import numpy as np
import jax
import jax.numpy as jnp
from jax import lax
from jax.experimental import pallas as pl

N_FFT = 256
HOP = 128
E = 16
GH = 256
HE = 64
OUT = 64
B = 1024
L = 2048
FREQ = N_FFT // 2 + 1
FRAMES = 17
BT = 256
EG = 4
GW = EG * HE


def _dft_consts():
    n = np.arange(N_FFT)
    f = np.arange(FREQ)
    win = 0.5 * (1.0 - np.cos(2.0 * np.pi * n / N_FFT))
    ang = 2.0 * np.pi * np.outer(n, f) / N_FFT
    cw = (win[:, None] * np.cos(ang)).astype(np.float32)
    sw = (win[:, None] * np.sin(ang)).astype(np.float32)
    return cw, sw


def _gating_kernel(xp_ref, cw_ref, sw_ref, gw1_ref, gb1_ref, gw2_ref, gb2_ref,
                   wmat_ref):
    frames = jnp.concatenate(
        [xp_ref[:, t * HOP:t * HOP + N_FFT] for t in range(FRAMES)], axis=0)
    re = jnp.dot(frames, cw_ref[...], preferred_element_type=jnp.float32)
    im = jnp.dot(frames, sw_ref[...], preferred_element_type=jnp.float32)
    mag = jnp.sqrt(re * re + im * im)
    pooled = mag[0:BT]
    for t in range(1, FRAMES):
        pooled = pooled + mag[t * BT:(t + 1) * BT]
    pooled = pooled * jnp.float32(1.0 / FRAMES)
    h = jnp.maximum(
        jnp.dot(pooled, gw1_ref[...], preferred_element_type=jnp.float32)
        + gb1_ref[...], 0.0)
    logits = jnp.dot(h, gw2_ref[...], preferred_element_type=jnp.float32) \
        + gb2_ref[...]
    ii = lax.broadcasted_iota(jnp.int32, (BT, E), 1)
    m1 = jnp.max(logits, axis=1, keepdims=True)
    i1 = jnp.min(jnp.where(logits == m1, ii, E), axis=1, keepdims=True)
    oh1 = ii == i1
    l2 = jnp.where(oh1, jnp.float32(-1e30), logits)
    m2 = jnp.max(l2, axis=1, keepdims=True)
    i2 = jnp.min(jnp.where(l2 == m2, ii, E), axis=1, keepdims=True)
    oh2 = ii == i2
    t = jnp.exp(m2 - m1)
    w1 = 1.0 / (1.0 + t)
    w2 = t * w1
    wmat_ref[...] = jnp.where(oh1, w1, 0.0) + jnp.where(oh2, w2, 0.0)


def _expert_kernel(x_ref, w1_ref, b1_ref, sel_ref, w2_ref, b2_ref, wmat_ref,
                   out_ref):
    g = pl.program_id(0)
    eh = jnp.maximum(
        jnp.dot(x_ref[...], w1_ref[...], preferred_element_type=jnp.float32)
        + b1_ref[...], 0.0)
    wexp = jnp.dot(wmat_ref[...], sel_ref[0],
                   preferred_element_type=jnp.float32)
    ehs = (eh * wexp).astype(jnp.bfloat16)
    contrib = jnp.dot(ehs, w2_ref[...], preferred_element_type=jnp.float32)

    @pl.when(g == 0)
    def _():
        out_ref[...] = jnp.dot(wmat_ref[...], b2_ref[...],
                               preferred_element_type=jnp.float32)

    out_ref[...] += contrib


def kernel(x, gW1, gb1, gW2, gb2, We1, be1, We2, be2):
    cw, sw = _dft_consts()
    cw = jnp.asarray(cw)
    sw = jnp.asarray(sw)
    xp = jnp.pad(x, ((0, 0), (HOP, HOP)), mode='reflect')

    wmat = pl.pallas_call(
        _gating_kernel,
        grid=(B // BT,),
        in_specs=[
            pl.BlockSpec((BT, L + 2 * HOP), lambda i: (i, 0)),
            pl.BlockSpec((N_FFT, FREQ), lambda i: (0, 0)),
            pl.BlockSpec((N_FFT, FREQ), lambda i: (0, 0)),
            pl.BlockSpec((FREQ, GH), lambda i: (0, 0)),
            pl.BlockSpec((1, GH), lambda i: (0, 0)),
            pl.BlockSpec((GH, E), lambda i: (0, 0)),
            pl.BlockSpec((1, E), lambda i: (0, 0)),
        ],
        out_specs=pl.BlockSpec((BT, E), lambda i: (i, 0)),
        out_shape=jax.ShapeDtypeStruct((B, E), jnp.float32),
    )(xp, cw, sw, gW1, gb1.reshape(1, GH), gW2, gb2.reshape(1, E))

    x16 = x.astype(jnp.bfloat16)
    w1cat = We1.transpose(1, 0, 2).reshape(L, E * HE).astype(jnp.bfloat16)
    b1cat = be1.reshape(1, E * HE)
    w2stack = We2.reshape(E * HE, OUT).astype(jnp.bfloat16)
    sel = np.zeros((E // EG, E, GW), dtype=np.float32)
    for g in range(E // EG):
        for j in range(EG):
            sel[g, EG * g + j, HE * j:HE * (j + 1)] = 1.0
    sel = jnp.asarray(sel)

    out = pl.pallas_call(
        _expert_kernel,
        grid=(E // EG,),
        in_specs=[
            pl.BlockSpec((B, L), lambda g: (0, 0)),
            pl.BlockSpec((L, GW), lambda g: (0, g)),
            pl.BlockSpec((1, GW), lambda g: (0, g)),
            pl.BlockSpec((1, E, GW), lambda g: (g, 0, 0)),
            pl.BlockSpec((GW, OUT), lambda g: (g, 0)),
            pl.BlockSpec((E, OUT), lambda g: (0, 0)),
            pl.BlockSpec((B, E), lambda g: (0, 0)),
        ],
        out_specs=pl.BlockSpec((B, OUT), lambda g: (0, 0)),
        out_shape=jax.ShapeDtypeStruct((B, OUT), jnp.float32),
    )(x16, w1cat, b1cat, sel, w2stack, be2, wmat)
    return out


# speedup vs baseline: 4.4684x; 1.3700x over previous
import numpy as np
import jax
import jax.numpy as jnp
from jax import lax
from jax.experimental import pallas as pl

N_FFT = 256
HOP = 128
E = 16
GH = 256
HE = 64
OUT = 64
B = 1024
L = 2048
FREQ = N_FFT // 2 + 1
FRAMES = 17
BT = 256
EG = 4
GW = EG * HE


def _dft_consts():
    n = np.arange(N_FFT)
    f = np.arange(FREQ)
    win = 0.5 * (1.0 - np.cos(2.0 * np.pi * n / N_FFT))
    ang = 2.0 * np.pi * np.outer(n, f) / N_FFT
    cw = win[:, None] * np.cos(ang)
    sw = win[:, None] * np.sin(ang)
    e0 = np.zeros((N_FFT, FREQ))
    e0[0:HOP] += cw[HOP:]
    e0[1:HOP + 1] += cw[0:HOP][::-1]
    s0 = np.zeros((N_FFT, FREQ))
    s0[0:HOP] += sw[HOP:]
    s0[1:HOP + 1] += sw[0:HOP][::-1]
    e16 = np.zeros((N_FFT, FREQ))
    e16[HOP:] += cw[0:HOP]
    e16[HOP - 1:N_FFT - 1] += cw[HOP:][::-1]
    s16 = np.zeros((N_FFT, FREQ))
    s16[HOP:] += sw[0:HOP]
    s16[HOP - 1:N_FFT - 1] += sw[HOP:][::-1]
    f32 = np.float32
    return cw.astype(f32), sw.astype(f32), e0.astype(f32), s0.astype(f32), \
        e16.astype(f32), s16.astype(f32)


def _mag(re, im):
    s = re * re + im * im
    return s * lax.rsqrt(s + jnp.float32(1e-30))


def _gating_kernel(x_ref, cw_ref, sw_ref, cw0_ref, sw0_ref, cw16_ref,
                   sw16_ref, gw1_ref, gb1_ref, gw2_ref, gb2_ref,
                   wmat_ref, x16_ref):
    x16_ref[...] = x_ref[...].astype(jnp.bfloat16)
    frames = jnp.concatenate(
        [x_ref[:, s * HOP:s * HOP + N_FFT] for s in range(FRAMES - 2)], axis=0)
    re = jnp.dot(frames, cw_ref[...], preferred_element_type=jnp.float32)
    im = jnp.dot(frames, sw_ref[...], preferred_element_type=jnp.float32)
    mag = _mag(re, im)
    fr0 = x_ref[:, 0:N_FFT]
    pooled = _mag(
        jnp.dot(fr0, cw0_ref[...], preferred_element_type=jnp.float32),
        jnp.dot(fr0, sw0_ref[...], preferred_element_type=jnp.float32))
    fr16 = x_ref[:, L - N_FFT:L]
    pooled += _mag(
        jnp.dot(fr16, cw16_ref[...], preferred_element_type=jnp.float32),
        jnp.dot(fr16, sw16_ref[...], preferred_element_type=jnp.float32))
    for t in range(FRAMES - 2):
        pooled = pooled + mag[t * BT:(t + 1) * BT]
    pooled = pooled * jnp.float32(1.0 / FRAMES)
    h = jnp.maximum(
        jnp.dot(pooled, gw1_ref[...], preferred_element_type=jnp.float32)
        + gb1_ref[...], 0.0)
    logits = jnp.dot(h, gw2_ref[...], preferred_element_type=jnp.float32) \
        + gb2_ref[...]
    ii = lax.broadcasted_iota(jnp.int32, (BT, E), 1)
    m1 = jnp.max(logits, axis=1, keepdims=True)
    i1 = jnp.min(jnp.where(logits == m1, ii, E), axis=1, keepdims=True)
    oh1 = ii == i1
    l2 = jnp.where(oh1, jnp.float32(-1e30), logits)
    m2 = jnp.max(l2, axis=1, keepdims=True)
    i2 = jnp.min(jnp.where(l2 == m2, ii, E), axis=1, keepdims=True)
    oh2 = ii == i2
    t = jnp.exp(m2 - m1)
    w1 = 1.0 / (1.0 + t)
    w2 = t * w1
    wmat_ref[...] = jnp.where(oh1, w1, 0.0) + jnp.where(oh2, w2, 0.0)


def _expert_kernel(x_ref, w1_ref, b1_ref, sel_ref, w2_ref, b2_ref, wmat_ref,
                   out_ref):
    g = pl.program_id(0)
    eh = jnp.maximum(
        jnp.dot(x_ref[...], w1_ref[...], preferred_element_type=jnp.float32)
        + b1_ref[...], 0.0)
    wexp = jnp.dot(wmat_ref[...], sel_ref[0],
                   preferred_element_type=jnp.float32)
    ehs = (eh * wexp).astype(jnp.bfloat16)
    contrib = jnp.dot(ehs, w2_ref[...], preferred_element_type=jnp.float32)

    @pl.when(g == 0)
    def _():
        out_ref[...] = jnp.dot(wmat_ref[...], b2_ref[...],
                               preferred_element_type=jnp.float32)

    out_ref[...] += contrib


def kernel(x, gW1, gb1, gW2, gb2, We1, be1, We2, be2):
    cw, sw, cw0, sw0, cw16, sw16 = map(jnp.asarray, _dft_consts())

    mat_spec = pl.BlockSpec((N_FFT, FREQ), lambda i: (0, 0))
    wmat, x16 = pl.pallas_call(
        _gating_kernel,
        grid=(B // BT,),
        in_specs=[
            pl.BlockSpec((BT, L), lambda i: (i, 0)),
            mat_spec, mat_spec, mat_spec, mat_spec, mat_spec, mat_spec,
            pl.BlockSpec((FREQ, GH), lambda i: (0, 0)),
            pl.BlockSpec((1, GH), lambda i: (0, 0)),
            pl.BlockSpec((GH, E), lambda i: (0, 0)),
            pl.BlockSpec((1, E), lambda i: (0, 0)),
        ],
        out_specs=[
            pl.BlockSpec((BT, E), lambda i: (i, 0)),
            pl.BlockSpec((BT, L), lambda i: (i, 0)),
        ],
        out_shape=[
            jax.ShapeDtypeStruct((B, E), jnp.float32),
            jax.ShapeDtypeStruct((B, L), jnp.bfloat16),
        ],
    )(x, cw, sw, cw0, sw0, cw16, sw16, gW1, gb1.reshape(1, GH), gW2,
      gb2.reshape(1, E))
    w1cat = We1.transpose(1, 0, 2).reshape(L, E * HE).astype(jnp.bfloat16)
    b1cat = be1.reshape(1, E * HE)
    w2stack = We2.reshape(E * HE, OUT).astype(jnp.bfloat16)
    sel = np.zeros((E // EG, E, GW), dtype=np.float32)
    for g in range(E // EG):
        for j in range(EG):
            sel[g, EG * g + j, HE * j:HE * (j + 1)] = 1.0
    sel = jnp.asarray(sel)

    out = pl.pallas_call(
        _expert_kernel,
        grid=(E // EG,),
        in_specs=[
            pl.BlockSpec((B, L), lambda g: (0, 0)),
            pl.BlockSpec((L, GW), lambda g: (0, g)),
            pl.BlockSpec((1, GW), lambda g: (0, g)),
            pl.BlockSpec((1, E, GW), lambda g: (g, 0, 0)),
            pl.BlockSpec((GW, OUT), lambda g: (g, 0)),
            pl.BlockSpec((E, OUT), lambda g: (0, 0)),
            pl.BlockSpec((B, E), lambda g: (0, 0)),
        ],
        out_specs=pl.BlockSpec((B, OUT), lambda g: (0, 0)),
        out_shape=jax.ShapeDtypeStruct((B, OUT), jnp.float32),
    )(x16, w1cat, b1cat, sel, w2stack, be2, wmat)
    return out
